# Initial kernel scaffold; baseline (speedup 1.0000x reference)
import jax
import jax.numpy as jnp
from jax.experimental import pallas as pl


def kernel(predictions, targets_in_grid, targets_masks, anchors):
    raise NotImplementedError("write your pallas kernel here")


# speedup vs baseline: 2.1069x; 2.1069x over previous
import functools

import jax
import jax.numpy as jnp
import numpy as np
from jax.experimental import pallas as pl
from jax.experimental.pallas import tpu as pltpu

ANCHOR_GAIN = 2.0
EPS = 1e-7


def _bce(z, t):
    return jnp.maximum(z, 0.0) - z * t + jnp.log1p(jnp.exp(-jnp.abs(z)))


def _atan_pos(x):
    inv = x > 1.0
    r = jnp.where(inv, 1.0 / jnp.maximum(x, 1e-30), x)
    z = r * r
    p = 0.99997726 + z * (-0.33262347 + z * (0.19354346 + z * (
        -0.11643287 + z * (0.05265332 + z * -0.01172120))))
    a = r * p
    return jnp.where(inv, (np.pi / 2.0) - a, a)


def _loss_kernel(pred_ref, tgt_ref, mask_ref, anchors_ref, out_ref, acc_ref,
                 *, n_blocks, n_obj, n_cls, blk_ba, n_anchors):
    step = pl.program_id(0)

    @pl.when(step == 0)
    def _init():
        for k in range(4):
            acc_ref[k] = jnp.float32(0.0)

    s_box = jnp.float32(0.0)
    s_np = jnp.float32(0.0)
    s_obj = jnp.float32(0.0)
    s_cls = jnp.float32(0.0)
    for j in range(blk_ba):
        x = pred_ref[j]
        t = tgt_ref[j]
        m = mask_ref[j].astype(jnp.float32)

        a = (step * blk_ba + j) % n_anchors
        aw = anchors_ref[a, 2]
        ah = anchors_ref[a, 3]

        bce = _bce(x, t)
        row = jax.lax.broadcasted_iota(jnp.int32, bce.shape, 0)
        is_obj = (row == 4).astype(jnp.float32)
        is_cls = (row >= 5).astype(jnp.float32)
        s_obj = s_obj + jnp.sum(bce * is_obj)
        s_cls = s_cls + jnp.sum(bce * (is_cls * m))
        s_np = s_np + jnp.sum(m)

        G = ANCHOR_GAIN
        sig0 = jax.nn.sigmoid(x[0:1, :])
        sig1 = jax.nn.sigmoid(x[1:2, :])
        sig2 = jax.nn.sigmoid(x[2:3, :])
        sig3 = jax.nn.sigmoid(x[3:4, :])
        px = sig0 * G - (G - 1.0) / 2.0
        py = sig1 * G - (G - 1.0) / 2.0
        pw = (sig2 * G) ** 2 * aw
        ph = (sig3 * G) ** 2 * ah
        tx = t[0:1, :]
        ty = t[1:2, :]
        tw = t[2:3, :]
        th = t[3:4, :]

        b1x1 = px - pw * 0.5
        b1x2 = px + pw * 0.5
        b1y1 = py - ph * 0.5
        b1y2 = py + ph * 0.5
        b2x1 = tx - tw * 0.5
        b2x2 = tx + tw * 0.5
        b2y1 = ty - th * 0.5
        b2y2 = ty + th * 0.5
        inter = (jnp.clip(jnp.minimum(b1x2, b2x2) - jnp.maximum(b1x1, b2x1), 0.0)
                 * jnp.clip(jnp.minimum(b1y2, b2y2) - jnp.maximum(b1y1, b2y1), 0.0))
        union = pw * ph + tw * th - inter + EPS
        iou = inter / union
        cw = jnp.maximum(b1x2, b2x2) - jnp.minimum(b1x1, b2x1)
        ch = jnp.maximum(b1y2, b2y2) - jnp.minimum(b1y1, b2y1)
        c2 = cw * cw + ch * ch + EPS
        rho2 = (tx - px) ** 2 + (ty - py) ** 2
        v = (4.0 / np.pi ** 2) * (_atan_pos(tw / (th + EPS))
                                  - _atan_pos(pw / (ph + EPS))) ** 2
        alpha = v / (v - iou + 1.0 + EPS)
        ciou = iou - (rho2 / c2 + v * alpha)
        s_box = s_box + jnp.sum((1.0 - ciou) * m)

    acc_ref[0] += s_box
    acc_ref[1] += s_np
    acc_ref[2] += s_obj
    acc_ref[3] += s_cls

    @pl.when(step == n_blocks - 1)
    def _final():
        n_pos = jnp.maximum(acc_ref[1], 1.0)
        out_ref[0] = (acc_ref[0] / n_pos
                      + acc_ref[2] / jnp.float32(n_obj)
                      + acc_ref[3] / (n_pos * jnp.float32(n_cls)))


@jax.jit
def _yolo_loss(predictions, targets_in_grid, targets_masks, anchors):
    B, A, F, H, W = predictions.shape
    BA, HW = B * A, H * W
    blk_ba = 2
    n_blocks = BA // blk_ba
    pred = predictions.reshape(BA, F, HW)
    tgt = targets_in_grid.reshape(BA, F, HW)
    mask = targets_masks.reshape(BA, 1, HW)

    out = pl.pallas_call(
        functools.partial(_loss_kernel, n_blocks=n_blocks, n_obj=BA * HW,
                          n_cls=F - 5, blk_ba=blk_ba, n_anchors=A),
        grid=(n_blocks,),
        in_specs=[
            pl.BlockSpec((blk_ba, F, HW), lambda i: (i, 0, 0)),
            pl.BlockSpec((blk_ba, F, HW), lambda i: (i, 0, 0)),
            pl.BlockSpec((blk_ba, 1, HW), lambda i: (i, 0, 0)),
            pl.BlockSpec(memory_space=pltpu.SMEM),
        ],
        out_specs=pl.BlockSpec(memory_space=pltpu.SMEM),
        out_shape=jax.ShapeDtypeStruct((1,), jnp.float32),
        scratch_shapes=[pltpu.SMEM((4,), jnp.float32)],
    )(pred, tgt, mask, anchors)
    return out[0]


def kernel(predictions, targets_in_grid, targets_masks, anchors):
    return _yolo_loss(predictions, targets_in_grid, targets_masks, anchors)
